# Initial kernel scaffold; baseline (speedup 1.0000x reference)
import jax
import jax.numpy as jnp
from jax.experimental import pallas as pl


def kernel(sen, asp_position):
    raise NotImplementedError("write your pallas kernel here")


# speedup vs baseline: 5.8349x; 5.8349x over previous
import math

import jax
import jax.numpy as jnp
import numpy as np
from jax.experimental import pallas as pl
from jax.experimental.pallas import tpu as pltpu

D_MODEL = 2048
BS = 256


def _pe_table(n_rows: int) -> jnp.ndarray:
    position = np.arange(n_rows, dtype=np.float32)[:, None]
    div_term = np.exp(
        np.arange(0.0, D_MODEL, 2, dtype=np.float32) * (-math.log(10000.0) / D_MODEL)
    )
    pe = np.zeros((n_rows, D_MODEL), dtype=np.float32)
    pe[:, 0::2] = np.sin(position * div_term)
    pe[:, 1::2] = np.cos(position * div_term)
    return jnp.asarray(pe)


def _add_body(sen_ref, pe_ref, out_ref):
    out_ref[...] = sen_ref[...] + pe_ref[...][None, :, :]


def _gather_body(asp_ref, sen_ref, pe_ref, out_ref):
    b = pl.program_id(0)
    row = asp_ref[b, 0]
    sub = row % 8
    val = sen_ref[0, pl.ds(sub, 1), :] + pe_ref[pl.ds(sub, 1), :]
    valid = row < asp_ref[b, 1]
    out_ref[0, :, :] = jnp.where(valid, val, jnp.ones((), dtype=jnp.float32))


def kernel(sen, asp_position):
    B, S, D = sen.shape
    pe = _pe_table(S)

    nj = S // BS
    sen_embed = pl.pallas_call(
        _add_body,
        grid=(nj, B),
        in_specs=[
            pl.BlockSpec((1, BS, D), lambda j, b: (b, j, 0)),
            pl.BlockSpec((BS, D), lambda j, b: (j, 0)),
        ],
        out_specs=pl.BlockSpec((1, BS, D), lambda j, b: (b, j, 0)),
        out_shape=jax.ShapeDtypeStruct((B, S, D), jnp.float32),
    )(sen, pe)

    asp_flat = pl.pallas_call(
        _gather_body,
        grid_spec=pltpu.PrefetchScalarGridSpec(
            num_scalar_prefetch=1,
            grid=(B,),
            in_specs=[
                pl.BlockSpec((1, 8, D), lambda b, asp: (b, asp[b, 0] // 8, 0)),
                pl.BlockSpec((8, D), lambda b, asp: (asp[b, 0] // 8, 0)),
            ],
            out_specs=pl.BlockSpec((1, 1, D), lambda b, asp: (b, 0, 0)),
        ),
        out_shape=jax.ShapeDtypeStruct((B, 1, D), jnp.float32),
    )(asp_position, sen, pe)

    return sen_embed, asp_flat
